# baseline (speedup 1.0000x reference)
import functools

import jax
import jax.numpy as jnp
from jax import lax
from jax.experimental import pallas as pl
from jax.experimental.pallas import tpu as pltpu
from jax.experimental.pallas import tpu_sc as plsc

N = 10000
NP = 10240
D = 256
HF = 128
E = 160000
EP = 163840
G = 64
C = 2
NTILE = 16
NCHUNK = EP // (NTILE * 128)
HCHUNK = NCHUNK // 2


def _sc_mesh():
    return plsc.VectorSubcoreMesh(core_axis_name="c", subcore_axis_name="s")


@functools.partial(
    pl.kernel,
    mesh=_sc_mesh(),
    out_type=jax.ShapeDtypeStruct((2, NP, 128), jnp.float32),
    scratch_types=[
        pltpu.VMEM((HCHUNK, 128), jnp.int32),
        pltpu.VMEM((128, 128), jnp.float32),
        pltpu.VMEM_SHARED((NP, 128), jnp.float32),
    ],
)
def _count_kernel(dst_hbm, zeros_hbm, ones_hbm, out_hbm, idx_v, ones_v, acc_sh):
    cid = lax.axis_index("c")
    sid = lax.axis_index("s")

    pltpu.sync_copy(ones_hbm, ones_v)

    @pl.when(sid == 0)
    def _():
        pltpu.sync_copy(zeros_hbm, acc_sh)

    plsc.subcore_barrier()

    pltpu.sync_copy(dst_hbm.at[sid, pl.ds(cid * HCHUNK, HCHUNK)], idx_v)

    def _chunk(j, c):
        pltpu.sync_copy(ones_v, acc_sh.at[idx_v.at[j]], add=True)
        return c

    lax.fori_loop(0, HCHUNK, _chunk, 0)
    plsc.subcore_barrier()

    @pl.when(sid == 0)
    def _():
        pltpu.sync_copy(acc_sh, out_hbm.at[cid])


@functools.partial(
    pl.kernel,
    mesh=_sc_mesh(),
    out_type=jax.ShapeDtypeStruct((2, NP, HF), jnp.float32),
    scratch_types=[
        pltpu.VMEM((HCHUNK, 128), jnp.int32),
        pltpu.VMEM((HCHUNK, 128), jnp.int32),
        pltpu.VMEM((128, HF), jnp.float32),
        pltpu.VMEM((128, HF), jnp.float32),
        pltpu.VMEM_SHARED((NP, HF), jnp.float32),
        pltpu.SemaphoreType.DMA,
        pltpu.SemaphoreType.DMA,
        pltpu.SemaphoreType.DMA,
        pltpu.SemaphoreType.DMA,
    ],
)
def _prop_kernel(y_hbm, src_hbm, dst_hbm, zeros_hbm, out_hbm,
                 src_v, dst_v, buf0, buf1, acc_sh, g0, g1, s0, s1):
    cid = lax.axis_index("c")
    sid = lax.axis_index("s")

    @pl.when(sid == 0)
    def _():
        pltpu.sync_copy(zeros_hbm, acc_sh)

    plsc.subcore_barrier()

    def _gather(j, buf, sem):
        return pltpu.async_copy(y_hbm.at[src_v.at[j]], buf, sem)

    def _gather_wait(j, buf, sem):
        pltpu.make_async_copy(y_hbm.at[src_v.at[j]], buf, sem).wait()

    def _scat(j, buf, sem):
        return pltpu.async_copy(buf, acc_sh.at[dst_v.at[j]], sem, add=True)

    def _scat_wait(j, buf, sem):
        pltpu.make_async_copy(buf, acc_sh.at[dst_v.at[j]], sem).wait()

    for h in range(2):
        pltpu.sync_copy(src_hbm.at[cid, sid, pl.ds(h * HCHUNK, HCHUNK)], src_v)
        pltpu.sync_copy(dst_hbm.at[sid, pl.ds(h * HCHUNK, HCHUNK)], dst_v)

        _gather(0, buf0, g0)
        _gather(1, buf1, g1)

        def _body(k, c):
            j0 = 2 * k
            j1 = 2 * k + 1
            _gather_wait(j0, buf0, g0)
            _scat(j0, buf0, s0)
            _gather_wait(j1, buf1, g1)
            _scat(j1, buf1, s1)

            @pl.when(j0 + 2 < HCHUNK)
            def _():
                _scat_wait(j0, buf0, s0)
                _gather(j0 + 2, buf0, g0)

            @pl.when(j1 + 2 < HCHUNK)
            def _():
                _scat_wait(j1, buf1, s1)
                _gather(j1 + 2, buf1, g1)

            return c

        lax.fori_loop(0, HCHUNK // 2, _body, 0)
        _scat_wait(HCHUNK - 2, buf0, s0)
        _scat_wait(HCHUNK - 1, buf1, s1)
    plsc.subcore_barrier()

    @pl.when(sid == 0)
    def _():
        pltpu.sync_copy(acc_sh, out_hbm.at[cid])


_BN = 1024
_NBLK = NP // _BN


def _dinv_from(cnt_ref):
    cnt = cnt_ref[0, :, 0:1] + cnt_ref[1, :, 0:1]
    return lax.rsqrt(cnt + 1.0)


def _mm_first_body(x_ref, w_ref, cnt_ref, out_ref):
    dinv = _dinv_from(cnt_ref)
    y = jnp.dot(x_ref[...], w_ref[...],
                preferred_element_type=jnp.float32,
                precision=lax.Precision.HIGHEST) * dinv
    out_ref[0] = y[:, :HF]
    out_ref[1] = y[:, HF:]


def _mm_mid_body(agg_ref, y_ref, w_ref, b_ref, cnt_ref, out_ref):
    dinv = _dinv_from(cnt_ref)
    h = jnp.concatenate([agg_ref[0], agg_ref[1]], axis=1)
    yp = jnp.concatenate([y_ref[0], y_ref[1]], axis=1)
    z = (h + yp) * dinv + b_ref[...]
    z = jnp.where(z >= 0, z, 0.01 * z)
    y = jnp.dot(z, w_ref[...],
                preferred_element_type=jnp.float32,
                precision=lax.Precision.HIGHEST) * dinv
    out_ref[0] = y[:, :HF]
    out_ref[1] = y[:, HF:]


_BNP = 512
_NBP = NP // _BNP


def _pool_body(agg_ref, y_ref, b_ref, cnt_ref, batch_ref, wl_ref, bl_ref,
               out_ref, acc_ref, gcnt_ref):
    i = pl.program_id(0)

    @pl.when(i == 0)
    def _():
        acc_ref[...] = jnp.zeros_like(acc_ref)
        gcnt_ref[...] = jnp.zeros_like(gcnt_ref)

    dinv = _dinv_from(cnt_ref)
    h = (jnp.concatenate([agg_ref[0], agg_ref[1]], axis=1)
         + jnp.concatenate([y_ref[0], y_ref[1]], axis=1)) * dinv + b_ref[...]
    bt = batch_ref[0, 0, :]
    onehot = (bt[:, None] == lax.broadcasted_iota(jnp.int32, (_BNP, G), 1))
    onehot = onehot.astype(jnp.float32)
    acc_ref[...] += lax.dot_general(
        onehot, h, (((0,), (0,)), ((), ())),
        preferred_element_type=jnp.float32,
        precision=lax.Precision.HIGHEST)
    gcnt_ref[...] += jnp.sum(onehot, axis=0, keepdims=True)

    @pl.when(i == _NBP - 1)
    def _():
        pooled = acc_ref[...] / jnp.maximum(gcnt_ref[0, :], 1.0)[:, None]
        logits = jnp.dot(pooled, wl_ref[...],
                         preferred_element_type=jnp.float32,
                         precision=lax.Precision.HIGHEST) + bl_ref[...]
        m = jnp.max(logits, axis=1, keepdims=True)
        e = jnp.exp(logits - m)
        out_ref[...] = e / jnp.sum(e, axis=1, keepdims=True)


def _mm_first(x_p, w, counts2):
    return pl.pallas_call(
        _mm_first_body,
        grid=(_NBLK,),
        in_specs=[
            pl.BlockSpec((_BN, D), lambda i: (i, 0)),
            pl.BlockSpec((D, D), lambda i: (0, 0)),
            pl.BlockSpec((2, _BN, 128), lambda i: (0, i, 0)),
        ],
        out_specs=pl.BlockSpec((2, _BN, HF), lambda i: (0, i, 0)),
        out_shape=jax.ShapeDtypeStruct((2, NP, HF), jnp.float32),
    )(x_p, w, counts2)


def _mm_mid(agg, y, w, b2d, counts2):
    return pl.pallas_call(
        _mm_mid_body,
        grid=(_NBLK,),
        in_specs=[
            pl.BlockSpec((2, _BN, HF), lambda i: (0, i, 0)),
            pl.BlockSpec((2, _BN, HF), lambda i: (0, i, 0)),
            pl.BlockSpec((D, D), lambda i: (0, 0)),
            pl.BlockSpec((1, D), lambda i: (0, 0)),
            pl.BlockSpec((2, _BN, 128), lambda i: (0, i, 0)),
        ],
        out_specs=pl.BlockSpec((2, _BN, HF), lambda i: (0, i, 0)),
        out_shape=jax.ShapeDtypeStruct((2, NP, HF), jnp.float32),
    )(agg, y, w, b2d, counts2)


def _pool(agg, y, b2d, counts2, batch3, wl, bl2d):
    return pl.pallas_call(
        _pool_body,
        grid=(_NBP,),
        in_specs=[
            pl.BlockSpec((2, _BNP, HF), lambda i: (0, i, 0)),
            pl.BlockSpec((2, _BNP, HF), lambda i: (0, i, 0)),
            pl.BlockSpec((1, D), lambda i: (0, 0)),
            pl.BlockSpec((2, _BNP, 128), lambda i: (0, i, 0)),
            pl.BlockSpec((1, 1, _BNP), lambda i: (i, 0, 0)),
            pl.BlockSpec((D, C), lambda i: (0, 0)),
            pl.BlockSpec((1, C), lambda i: (0, 0)),
        ],
        out_specs=pl.BlockSpec((G, C), lambda i: (0, 0)),
        out_shape=jax.ShapeDtypeStruct((G, C), jnp.float32),
        scratch_shapes=[
            pltpu.VMEM((G, D), jnp.float32),
            pltpu.VMEM((1, G), jnp.float32),
        ],
    )(agg, y, b2d, counts2, batch3, wl, bl2d)


def kernel(x, edge_index, batch, W1, b1, W2, b2, W3, b3, Wl, bl):
    src = edge_index[0].astype(jnp.int32)
    dst = edge_index[1].astype(jnp.int32)

    pad = EP - E
    ar = jnp.arange(pad, dtype=jnp.int32)
    pad_src = (ar * 37) % N
    pad_dst = N + ar % (NP - N)
    srcp = jnp.concatenate([src, pad_src]).reshape(NTILE, NCHUNK, 128)
    dstp = jnp.concatenate([dst, pad_dst]).reshape(NTILE, NCHUNK, 128)
    src2 = jnp.stack([srcp, srcp + NP])

    x_p = jnp.concatenate([x, jnp.zeros((NP - N, D), jnp.float32)], axis=0)
    batch3 = jnp.concatenate(
        [batch.astype(jnp.int32), jnp.full((NP - N,), G, jnp.int32)]
    ).reshape(_NBP, 1, _BNP)
    b1_2d = b1.reshape(1, D)
    b2_2d = b2.reshape(1, D)
    b3_2d = b3.reshape(1, D)
    bl_2d = bl.reshape(1, C)

    zeros_nf = jnp.zeros((NP, 128), jnp.float32)
    ones_f = jnp.ones((128, 128), jnp.float32)

    counts2 = _count_kernel(dstp, zeros_nf, ones_f)

    y1 = _mm_first(x_p, W1, counts2)
    agg1 = _prop_kernel(y1.reshape(2 * NP, HF), src2, dstp, zeros_nf)
    y2 = _mm_mid(agg1, y1, W2, b1_2d, counts2)
    agg2 = _prop_kernel(y2.reshape(2 * NP, HF), src2, dstp, zeros_nf)
    y3 = _mm_mid(agg2, y2, W3, b2_2d, counts2)
    agg3 = _prop_kernel(y3.reshape(2 * NP, HF), src2, dstp, zeros_nf)

    return _pool(agg3, y3, b3_2d, counts2, batch3, Wl, bl_2d)


# speedup vs baseline: 1.2258x; 1.2258x over previous
import functools

import jax
import jax.numpy as jnp
from jax import lax
from jax.experimental import pallas as pl
from jax.experimental.pallas import tpu as pltpu
from jax.experimental.pallas import tpu_sc as plsc

N = 10000
NP = 10240
D = 256
HF = 128
E = 160000
EP = 163840
G = 64
C = 2
NTILE = 16
NCHUNK = EP // (NTILE * 128)
HCHUNK = NCHUNK // 2


def _sc_mesh():
    return plsc.VectorSubcoreMesh(core_axis_name="c", subcore_axis_name="s")


@functools.partial(
    pl.kernel,
    mesh=_sc_mesh(),
    out_type=jax.ShapeDtypeStruct((2, NP, 128), jnp.float32),
    scratch_types=[
        pltpu.VMEM((HCHUNK, 128), jnp.int32),
        pltpu.VMEM((128, 128), jnp.float32),
        pltpu.VMEM_SHARED((NP, 128), jnp.float32),
        pltpu.SemaphoreType.DMA,
    ],
)
def _count_kernel(dst_hbm, zeros_hbm, ones_hbm, out_hbm,
                  idx_v, ones_v, acc_sh, sem):
    cid = lax.axis_index("c")
    sid = lax.axis_index("s")

    pltpu.sync_copy(ones_hbm, ones_v)

    @pl.when(sid == 0)
    def _():
        pltpu.sync_copy(zeros_hbm, acc_sh)

    plsc.subcore_barrier()

    pltpu.sync_copy(dst_hbm.at[sid, pl.ds(cid * HCHUNK, HCHUNK)], idx_v)

    def _chunk(j, c):
        pltpu.async_copy(ones_v, acc_sh.at[idx_v.at[j]], sem, add=True)
        return c

    lax.fori_loop(0, HCHUNK, _chunk, 0)

    def _drain(j, c):
        pltpu.make_async_copy(ones_v, acc_sh.at[idx_v.at[j]], sem).wait()
        return c

    lax.fori_loop(0, HCHUNK, _drain, 0)
    plsc.subcore_barrier()

    @pl.when(sid == 0)
    def _():
        pltpu.sync_copy(acc_sh, out_hbm.at[cid])


@functools.partial(
    pl.kernel,
    mesh=_sc_mesh(),
    out_type=jax.ShapeDtypeStruct((2, NP, HF), jnp.float32),
    scratch_types=[
        pltpu.VMEM((HCHUNK, 128), jnp.int32),
        pltpu.VMEM((HCHUNK, 128), jnp.int32),
        pltpu.VMEM((128, HF), jnp.float32),
        pltpu.VMEM((128, HF), jnp.float32),
        pltpu.VMEM_SHARED((NP, HF), jnp.float32),
        pltpu.SemaphoreType.DMA,
        pltpu.SemaphoreType.DMA,
    ],
)
def _prop_kernel(y_hbm, src_hbm, dst_hbm, zeros_hbm, out_hbm,
                 src_v, dst_v, buf0, buf1, acc_sh, g0, g1):
    cid = lax.axis_index("c")
    sid = lax.axis_index("s")

    @pl.when(sid == 0)
    def _():
        pltpu.sync_copy(zeros_hbm, acc_sh)

    plsc.subcore_barrier()

    for h in range(2):
        pltpu.sync_copy(src_hbm.at[cid, sid, pl.ds(h * HCHUNK, HCHUNK)], src_v)
        pltpu.sync_copy(dst_hbm.at[sid, pl.ds(h * HCHUNK, HCHUNK)], dst_v)

        pltpu.async_copy(y_hbm.at[src_v.at[0]], buf0, g0)

        def _body(k, c):
            j0 = 2 * k
            j1 = 2 * k + 1
            pltpu.async_copy(y_hbm.at[src_v.at[j1]], buf1, g1)
            pltpu.make_async_copy(y_hbm.at[src_v.at[j0]], buf0, g0).wait()
            pltpu.sync_copy(buf0, acc_sh.at[dst_v.at[j0]], add=True)

            @pl.when(j1 + 1 < HCHUNK)
            def _():
                pltpu.async_copy(y_hbm.at[src_v.at[j1 + 1]], buf0, g0)

            pltpu.make_async_copy(y_hbm.at[src_v.at[j1]], buf1, g1).wait()
            pltpu.sync_copy(buf1, acc_sh.at[dst_v.at[j1]], add=True)
            return c

        lax.fori_loop(0, HCHUNK // 2, _body, 0)
    plsc.subcore_barrier()

    @pl.when(sid == 0)
    def _():
        pltpu.sync_copy(acc_sh, out_hbm.at[cid])


_BN = 1024
_NBLK = NP // _BN


def _dinv_from(cnt_ref):
    cnt = cnt_ref[0, :, 0:1] + cnt_ref[1, :, 0:1]
    return lax.rsqrt(cnt + 1.0)


def _mm_first_body(x_ref, w_ref, cnt_ref, out_ref):
    dinv = _dinv_from(cnt_ref)
    y = jnp.dot(x_ref[...], w_ref[...],
                preferred_element_type=jnp.float32,
                precision=lax.Precision.HIGHEST) * dinv
    out_ref[0] = y[:, :HF]
    out_ref[1] = y[:, HF:]


def _mm_mid_body(agg_ref, y_ref, w_ref, b_ref, cnt_ref, out_ref):
    dinv = _dinv_from(cnt_ref)
    h = jnp.concatenate([agg_ref[0], agg_ref[1]], axis=1)
    yp = jnp.concatenate([y_ref[0], y_ref[1]], axis=1)
    z = (h + yp) * dinv + b_ref[...]
    z = jnp.where(z >= 0, z, 0.01 * z)
    y = jnp.dot(z, w_ref[...],
                preferred_element_type=jnp.float32,
                precision=lax.Precision.HIGHEST) * dinv
    out_ref[0] = y[:, :HF]
    out_ref[1] = y[:, HF:]


_BNP = 512
_NBP = NP // _BNP


def _pool_body(agg_ref, y_ref, b_ref, cnt_ref, batch_ref, wl_ref, bl_ref,
               out_ref, acc_ref, gcnt_ref):
    i = pl.program_id(0)

    @pl.when(i == 0)
    def _():
        acc_ref[...] = jnp.zeros_like(acc_ref)
        gcnt_ref[...] = jnp.zeros_like(gcnt_ref)

    dinv = _dinv_from(cnt_ref)
    h = (jnp.concatenate([agg_ref[0], agg_ref[1]], axis=1)
         + jnp.concatenate([y_ref[0], y_ref[1]], axis=1)) * dinv + b_ref[...]
    bt = batch_ref[0, 0, :]
    onehot = (bt[:, None] == lax.broadcasted_iota(jnp.int32, (_BNP, G), 1))
    onehot = onehot.astype(jnp.float32)
    acc_ref[...] += lax.dot_general(
        onehot, h, (((0,), (0,)), ((), ())),
        preferred_element_type=jnp.float32,
        precision=lax.Precision.HIGHEST)
    gcnt_ref[...] += jnp.sum(onehot, axis=0, keepdims=True)

    @pl.when(i == _NBP - 1)
    def _():
        pooled = acc_ref[...] / jnp.maximum(gcnt_ref[0, :], 1.0)[:, None]
        logits = jnp.dot(pooled, wl_ref[...],
                         preferred_element_type=jnp.float32,
                         precision=lax.Precision.HIGHEST) + bl_ref[...]
        m = jnp.max(logits, axis=1, keepdims=True)
        e = jnp.exp(logits - m)
        out_ref[...] = e / jnp.sum(e, axis=1, keepdims=True)


def _mm_first(x_p, w, counts2):
    return pl.pallas_call(
        _mm_first_body,
        grid=(_NBLK,),
        in_specs=[
            pl.BlockSpec((_BN, D), lambda i: (i, 0)),
            pl.BlockSpec((D, D), lambda i: (0, 0)),
            pl.BlockSpec((2, _BN, 128), lambda i: (0, i, 0)),
        ],
        out_specs=pl.BlockSpec((2, _BN, HF), lambda i: (0, i, 0)),
        out_shape=jax.ShapeDtypeStruct((2, NP, HF), jnp.float32),
    )(x_p, w, counts2)


def _mm_mid(agg, y, w, b2d, counts2):
    return pl.pallas_call(
        _mm_mid_body,
        grid=(_NBLK,),
        in_specs=[
            pl.BlockSpec((2, _BN, HF), lambda i: (0, i, 0)),
            pl.BlockSpec((2, _BN, HF), lambda i: (0, i, 0)),
            pl.BlockSpec((D, D), lambda i: (0, 0)),
            pl.BlockSpec((1, D), lambda i: (0, 0)),
            pl.BlockSpec((2, _BN, 128), lambda i: (0, i, 0)),
        ],
        out_specs=pl.BlockSpec((2, _BN, HF), lambda i: (0, i, 0)),
        out_shape=jax.ShapeDtypeStruct((2, NP, HF), jnp.float32),
    )(agg, y, w, b2d, counts2)


def _pool(agg, y, b2d, counts2, batch3, wl, bl2d):
    return pl.pallas_call(
        _pool_body,
        grid=(_NBP,),
        in_specs=[
            pl.BlockSpec((2, _BNP, HF), lambda i: (0, i, 0)),
            pl.BlockSpec((2, _BNP, HF), lambda i: (0, i, 0)),
            pl.BlockSpec((1, D), lambda i: (0, 0)),
            pl.BlockSpec((2, _BNP, 128), lambda i: (0, i, 0)),
            pl.BlockSpec((1, 1, _BNP), lambda i: (i, 0, 0)),
            pl.BlockSpec((D, C), lambda i: (0, 0)),
            pl.BlockSpec((1, C), lambda i: (0, 0)),
        ],
        out_specs=pl.BlockSpec((G, C), lambda i: (0, 0)),
        out_shape=jax.ShapeDtypeStruct((G, C), jnp.float32),
        scratch_shapes=[
            pltpu.VMEM((G, D), jnp.float32),
            pltpu.VMEM((1, G), jnp.float32),
        ],
    )(agg, y, b2d, counts2, batch3, wl, bl2d)


def kernel(x, edge_index, batch, W1, b1, W2, b2, W3, b3, Wl, bl):
    src = edge_index[0].astype(jnp.int32)
    dst = edge_index[1].astype(jnp.int32)

    pad = EP - E
    ar = jnp.arange(pad, dtype=jnp.int32)
    pad_src = (ar * 37) % N
    pad_dst = N + ar % (NP - N)
    srcp = jnp.concatenate([src, pad_src]).reshape(NTILE, NCHUNK, 128)
    dstp = jnp.concatenate([dst, pad_dst]).reshape(NTILE, NCHUNK, 128)
    src2 = jnp.stack([srcp, srcp + NP])

    x_p = jnp.concatenate([x, jnp.zeros((NP - N, D), jnp.float32)], axis=0)
    batch3 = jnp.concatenate(
        [batch.astype(jnp.int32), jnp.full((NP - N,), G, jnp.int32)]
    ).reshape(_NBP, 1, _BNP)
    b1_2d = b1.reshape(1, D)
    b2_2d = b2.reshape(1, D)
    b3_2d = b3.reshape(1, D)
    bl_2d = bl.reshape(1, C)

    zeros_nf = jnp.zeros((NP, 128), jnp.float32)
    ones_f = jnp.ones((128, 128), jnp.float32)

    counts2 = _count_kernel(dstp, zeros_nf, ones_f)

    y1 = _mm_first(x_p, W1, counts2)
    agg1 = _prop_kernel(y1.reshape(2 * NP, HF), src2, dstp, zeros_nf)
    y2 = _mm_mid(agg1, y1, W2, b1_2d, counts2)
    agg2 = _prop_kernel(y2.reshape(2 * NP, HF), src2, dstp, zeros_nf)
    y3 = _mm_mid(agg2, y2, W3, b2_2d, counts2)
    agg3 = _prop_kernel(y3.reshape(2 * NP, HF), src2, dstp, zeros_nf)

    return _pool(agg3, y3, b3_2d, counts2, batch3, Wl, bl_2d)
